# baseline (speedup 1.0000x reference)
import functools

import jax
import jax.numpy as jnp
from jax import lax
from jax.experimental import pallas as pl
from jax.experimental.pallas import tpu as pltpu
from jax.experimental.pallas import tpu_sc as plsc


@functools.cache
def _make_gather(V, D, R, J):
    info = plsc.get_sparse_core_info()
    NC, NS = info.num_cores, info.num_subcores
    NW = NC * NS
    assert R % NW == 0
    r_per_w = R // NW
    K = 8
    assert r_per_w % K == 0
    n_chunks = r_per_w // K
    NBUF = 4
    G = 2
    assert G < NBUF <= n_chunks
    mesh = plsc.VectorSubcoreMesh(core_axis_name="c", subcore_axis_name="s")

    @functools.partial(
        pl.kernel,
        mesh=mesh,
        out_type=jax.ShapeDtypeStruct((R, J, D), jnp.float32),
        scratch_types=[
            pltpu.VMEM((r_per_w, J), jnp.int32),
            pltpu.VMEM((NBUF, K, J, D), jnp.float32),
            pltpu.SemaphoreType.DMA((NBUF,)),
            pltpu.SemaphoreType.DMA((NBUF,)),
        ],
        compiler_params=pltpu.CompilerParams(use_tc_tiling_on_sc=False),
    )
    def k(table_hbm, x_hbm, out_hbm, idx_v, rows_v, gsem, ssem):
        wid = lax.axis_index("s") * NC + lax.axis_index("c")
        r0 = wid * r_per_w
        pltpu.sync_copy(x_hbm.at[pl.ds(r0, r_per_w)], idx_v)

        def gathers_start(c, b):
            for u in range(K):
                pltpu.async_copy(
                    table_hbm.at[idx_v.at[c * K + u]], rows_v.at[b, u], gsem.at[b]
                )

        def gathers_wait(c, b):
            for u in range(K):
                pltpu.make_async_copy(
                    table_hbm.at[idx_v.at[c * K + u]], rows_v.at[b, u], gsem.at[b]
                ).wait()

        def store_start(c, b):
            pltpu.async_copy(
                rows_v.at[b], out_hbm.at[pl.ds(r0 + c * K, K)], ssem.at[b]
            )

        def store_wait(c, b):
            pltpu.make_async_copy(
                rows_v.at[b], out_hbm.at[pl.ds(r0 + c * K, K)], ssem.at[b]
            ).wait()

        for c in range(G):
            gathers_start(c, c)

        def body(c, carry):
            b = lax.rem(c, NBUF)
            j = c + G
            bj = lax.rem(j, NBUF)

            @pl.when(j < n_chunks)
            def _():
                @pl.when(j >= NBUF)
                def _():
                    store_wait(j - NBUF, bj)
                gathers_start(j, bj)

            gathers_wait(c, b)
            store_start(c, b)
            return carry

        lax.fori_loop(0, n_chunks, body, 0)

        for u in range(NBUF):
            c_last = n_chunks - NBUF + ((u - (n_chunks - NBUF)) % NBUF)
            store_wait(c_last, u)

    return k


def kernel(x, table):
    R, J = x.shape
    V, D = table.shape
    return _make_gather(V, D, R, J)(table, x)
